# baseline (speedup 1.0000x reference)
import jax
import jax.numpy as jnp
from jax.experimental import pallas as pl

N = 50000
D = 256
D_INNER = 1024
BN1 = 2000
BN2 = 1000


def _phi(z):
    return jnp.where(z > 0, z + 1.0, jnp.exp(z))


def _ln(xb, g, b, eps=1e-5):
    mu = jnp.mean(xb, axis=-1, keepdims=True)
    var = jnp.mean((xb - mu) ** 2, axis=-1, keepdims=True)
    return (xb - mu) * jax.lax.rsqrt(var + eps) * g + b


def _bdot(a, b):
    return jnp.dot(a.astype(jnp.bfloat16), b, preferred_element_type=jnp.float32)


def _pass1(x_ref, wk_ref, wv_ref, g_ref, b_ref, kv_ref, ksum_ref):
    i = pl.program_id(0)
    h = _ln(x_ref[...], g_ref[...], b_ref[...])
    hb = h.astype(jnp.bfloat16)
    k = jnp.dot(hb, wk_ref[...], preferred_element_type=jnp.float32)
    v = jnp.dot(hb, wv_ref[...], preferred_element_type=jnp.float32)
    phik = _phi(k)
    pkv = jax.lax.dot_general(
        phik.astype(jnp.bfloat16), v.astype(jnp.bfloat16),
        (((0,), (0,)), ((), ())),
        preferred_element_type=jnp.float32)
    pksum = jnp.sum(phik, axis=0, keepdims=True)

    @pl.when(i == 0)
    def _():
        kv_ref[...] = jnp.zeros_like(kv_ref)
        ksum_ref[...] = jnp.zeros_like(ksum_ref)

    kv_ref[...] += pkv
    ksum_ref[...] += pksum


def _pass2(x_ref, wq_ref, wo_ref, kv_ref, ksum_ref, g1_ref, b1_ref,
           w1_ref, bb1_ref, w2_ref, bb2_ref, g2_ref, b2_ref, out_ref):
    xb = x_ref[...]
    h = _ln(xb, g1_ref[...], b1_ref[...])
    q = _bdot(h, wq_ref[...])
    phiq = _phi(q)
    num = _bdot(phiq, kv_ref[...])
    den = jnp.sum(phiq * ksum_ref[...], axis=1, keepdims=True) + 1e-6
    attn = _bdot(num / den, wo_ref[...])
    x2 = xb + attn
    h2 = _ln(x2, g2_ref[...], b2_ref[...])
    inner = jax.nn.gelu(_bdot(h2, w1_ref[...]) + bb1_ref[...])
    mlp = _bdot(inner, w2_ref[...])
    out_ref[...] = x2 + mlp + bb2_ref[...]


def kernel(x, Wq, Wk, Wv, Wo, ln1_g, ln1_b, W1, b1, W2, b2, ln2_g, ln2_b):
    g1 = ln1_g.reshape(1, D)
    bt1 = ln1_b.reshape(1, D)
    g2 = ln2_g.reshape(1, D)
    bt2 = ln2_b.reshape(1, D)
    bb1 = b1.reshape(1, D_INNER)
    bb2 = b2.reshape(1, D)
    bf = jnp.bfloat16
    Wqb, Wkb, Wvb, Wob = (w.astype(bf) for w in (Wq, Wk, Wv, Wo))
    W1b, W2b = W1.astype(bf), W2.astype(bf)

    full = lambda shape: pl.BlockSpec(shape, lambda i: (0,) * len(shape))

    kv, ksum = pl.pallas_call(
        _pass1,
        grid=(N // BN1,),
        in_specs=[
            pl.BlockSpec((BN1, D), lambda i: (i, 0)),
            full((D, D)), full((D, D)), full((1, D)), full((1, D)),
        ],
        out_specs=[full((D, D)), full((1, D))],
        out_shape=[
            jax.ShapeDtypeStruct((D, D), jnp.float32),
            jax.ShapeDtypeStruct((1, D), jnp.float32),
        ],
    )(x, Wkb, Wvb, g1, bt1)

    kvb = kv.astype(bf)

    out = pl.pallas_call(
        _pass2,
        grid=(N // BN2,),
        in_specs=[
            pl.BlockSpec((BN2, D), lambda i: (i, 0)),
            full((D, D)), full((D, D)), full((D, D)), full((1, D)),
            full((1, D)), full((1, D)),
            full((D, D_INNER)), full((1, D_INNER)),
            full((D_INNER, D)), full((1, D)),
            full((1, D)), full((1, D)),
        ],
        out_specs=pl.BlockSpec((BN2, D), lambda i: (i, 0)),
        out_shape=jax.ShapeDtypeStruct((N, D), jnp.float32),
    )(x, Wqb, Wob, kvb, ksum, g1, bt1, W1b, bb1, W2b, bb2, g2, bt2)
    return out


# speedup vs baseline: 1.3003x; 1.3003x over previous
import jax
import jax.numpy as jnp
from jax.experimental import pallas as pl

N = 50000
D = 256
D_INNER = 1024
BN1 = 2000
BN2 = 2000


def _phi(z):
    one = jnp.asarray(1.0, z.dtype)
    return jnp.where(z > 0, z + one, jnp.exp(z))


def _ln(xb, g, b, eps=1e-5):
    mu = jnp.mean(xb, axis=-1, keepdims=True)
    ex2 = jnp.mean(xb * xb, axis=-1, keepdims=True)
    var = ex2 - mu * mu
    r = jax.lax.rsqrt(var + eps)
    return (xb - mu) * (r * g) + b


_GC1 = 0.7978845608028654
_GC2 = 0.7978845608028654 * 0.044715


def _gelu(t):
    tt = t * t
    u = t * (jnp.asarray(_GC1, t.dtype) + jnp.asarray(_GC2, t.dtype) * tt)
    th = jnp.tanh(u)
    r = jnp.asarray(0.5, t.dtype) * t
    return r + r * th


def _pass1(x_ref, wq_ref, wk_ref, wv_ref, g_ref, b_ref,
           kv_ref, ksum_ref, phiq_ref):
    i = pl.program_id(0)

    def _half(sl):
        h = _ln(x_ref[sl, :], g_ref[...], b_ref[...])
        hb = h.astype(jnp.bfloat16)
        q = jnp.dot(hb, wq_ref[...], preferred_element_type=jnp.float32)
        phiq_ref[sl, :] = _phi(q).astype(jnp.bfloat16)
        k = jnp.dot(hb, wk_ref[...], preferred_element_type=jnp.float32)
        v = jnp.dot(hb, wv_ref[...],
                    preferred_element_type=jnp.float32).astype(jnp.bfloat16)
        phik = _phi(k)
        pkv = jax.lax.dot_general(
            phik.astype(jnp.bfloat16), v, (((0,), (0,)), ((), ())),
            preferred_element_type=jnp.float32)
        pksum = jnp.sum(phik, axis=0, keepdims=True)
        return pkv, pksum

    pkv, pksum = _half(pl.ds(0, BN1))

    @pl.when(i == 0)
    def _():
        kv_ref[...] = jnp.zeros_like(kv_ref)
        ksum_ref[...] = jnp.zeros_like(ksum_ref)

    kv_ref[...] += pkv
    ksum_ref[...] += pksum


def _pass2(x_ref, phiq_ref, wo_ref, kv_ref, ksumt_ref, g2_ref, b2_ref,
           w1_ref, bb1_ref, w2_ref, bb2_ref, out_ref):
    def _half(sl):
        xb = x_ref[sl, :]
        phiq = phiq_ref[sl, :]
        num = jnp.dot(phiq, kv_ref[...], preferred_element_type=jnp.float32)
        den = jnp.dot(phiq, ksumt_ref[...],
                      preferred_element_type=jnp.float32) + 1e-6
        attn = jnp.dot((num / den).astype(jnp.bfloat16), wo_ref[...],
                       preferred_element_type=jnp.float32)
        x2 = xb + attn
        h2 = _ln(x2, g2_ref[...], b2_ref[...])
        t = jnp.dot(h2.astype(jnp.bfloat16), w1_ref[...],
                    preferred_element_type=jnp.float32)
        inner = _gelu(t.astype(jnp.bfloat16) + bb1_ref[...])
        mlp = jnp.dot(inner, w2_ref[...], preferred_element_type=jnp.float32)
        out_ref[sl, :] = x2 + mlp + bb2_ref[...]

    hb = BN2 // 2
    _half(pl.ds(0, hb))
    _half(pl.ds(hb, hb))


def kernel(x, Wq, Wk, Wv, Wo, ln1_g, ln1_b, W1, b1, W2, b2, ln2_g, ln2_b):
    bf = jnp.bfloat16
    g1 = ln1_g.reshape(1, D)
    bt1 = ln1_b.reshape(1, D)
    g2 = ln2_g.reshape(1, D)
    bt2 = ln2_b.reshape(1, D)
    bb1 = b1.reshape(1, D_INNER).astype(bf)
    bb2 = b2.reshape(1, D)

    full = lambda shape: pl.BlockSpec(shape, lambda i: (0,) * len(shape))

    kv, ksum, phiq = pl.pallas_call(
        _pass1,
        grid=(N // BN1,),
        in_specs=[
            pl.BlockSpec((BN1, D), lambda i: (i, 0)),
            full((D, D)), full((D, D)), full((D, D)),
            full((1, D)), full((1, D)),
        ],
        out_specs=[full((D, D)), full((1, D)),
                   pl.BlockSpec((BN1, D), lambda i: (i, 0))],
        out_shape=[
            jax.ShapeDtypeStruct((D, D), jnp.float32),
            jax.ShapeDtypeStruct((1, D), jnp.float32),
            jax.ShapeDtypeStruct((N, D), bf),
        ],
    )(x, Wq.astype(bf), Wk.astype(bf), Wv.astype(bf), g1, bt1)

    ksumt = ksum.reshape(D, 1).astype(bf)

    out = pl.pallas_call(
        _pass2,
        grid=(N // BN2,),
        in_specs=[
            pl.BlockSpec((BN2, D), lambda i: (i, 0)),
            pl.BlockSpec((BN2, D), lambda i: (i, 0)),
            full((D, D)), full((D, D)), full((D, 1)),
            full((1, D)), full((1, D)),
            full((D, D_INNER)), full((1, D_INNER)),
            full((D_INNER, D)), full((1, D)),
        ],
        out_specs=pl.BlockSpec((BN2, D), lambda i: (i, 0)),
        out_shape=jax.ShapeDtypeStruct((N, D), jnp.float32),
    )(x, phiq, Wo.astype(bf), kv.astype(bf), ksumt, g2, bt2,
      W1.astype(bf), bb1, W2.astype(bf), bb2)
    return out


# speedup vs baseline: 1.3143x; 1.0108x over previous
import jax
import jax.numpy as jnp
from jax.experimental import pallas as pl

N = 50000
D = 256
D_INNER = 1024
BN1 = 5000
BN2 = 5000


def _phi(z):
    one = jnp.asarray(1.0, z.dtype)
    return jnp.where(z > 0, z + one, jnp.exp(z))


def _ln(xb, g, b, eps=1e-5):
    mu = jnp.mean(xb, axis=-1, keepdims=True)
    ex2 = jnp.mean(xb * xb, axis=-1, keepdims=True)
    var = ex2 - mu * mu
    r = jax.lax.rsqrt(var + eps)
    return (xb - mu) * (r * g) + b


_GC1 = 0.7978845608028654
_GC2 = 0.7978845608028654 * 0.044715


def _gelu(t):
    tt = t * t
    u = t * (jnp.asarray(_GC1, t.dtype) + jnp.asarray(_GC2, t.dtype) * tt)
    th = jnp.tanh(u)
    r = jnp.asarray(0.5, t.dtype) * t
    return r + r * th


def _pass1(x_ref, wq_ref, wk_ref, wv_ref, g_ref, b_ref,
           kv_ref, ksum_ref, phiq_ref):
    i = pl.program_id(0)

    def _half(sl):
        h = _ln(x_ref[sl, :], g_ref[...], b_ref[...])
        hb = h.astype(jnp.bfloat16)
        q = jnp.dot(hb, wq_ref[...], preferred_element_type=jnp.float32)
        phiq_ref[sl, :] = _phi(q).astype(jnp.bfloat16)
        k = jnp.dot(hb, wk_ref[...], preferred_element_type=jnp.float32)
        v = jnp.dot(hb, wv_ref[...],
                    preferred_element_type=jnp.float32).astype(jnp.bfloat16)
        phik = _phi(k)
        pkv = jax.lax.dot_general(
            phik.astype(jnp.bfloat16), v, (((0,), (0,)), ((), ())),
            preferred_element_type=jnp.float32)
        pksum = jnp.sum(phik, axis=0, keepdims=True)
        return pkv, pksum

    pkv, pksum = _half(pl.ds(0, BN1))

    @pl.when(i == 0)
    def _():
        kv_ref[...] = jnp.zeros_like(kv_ref)
        ksum_ref[...] = jnp.zeros_like(ksum_ref)

    kv_ref[...] += pkv
    ksum_ref[...] += pksum


def _pass2(x_ref, phiq_ref, wo_ref, kv_ref, ksumt_ref, g2_ref, b2_ref,
           w1_ref, bb1_ref, w2_ref, bb2_ref, out_ref):
    def _half(sl):
        xb = x_ref[sl, :]
        phiq = phiq_ref[sl, :]
        num = jnp.dot(phiq, kv_ref[...], preferred_element_type=jnp.float32)
        den = jnp.dot(phiq, ksumt_ref[...],
                      preferred_element_type=jnp.float32) + 1e-6
        attn = jnp.dot((num / den).astype(jnp.bfloat16), wo_ref[...],
                       preferred_element_type=jnp.float32)
        x2 = xb + attn
        h2 = _ln(x2, g2_ref[...], b2_ref[...])
        t = jnp.dot(h2.astype(jnp.bfloat16), w1_ref[...],
                    preferred_element_type=jnp.float32)
        inner = _gelu(t.astype(jnp.bfloat16) + bb1_ref[...])
        mlp = jnp.dot(inner, w2_ref[...], preferred_element_type=jnp.float32)
        out_ref[sl, :] = x2 + mlp + bb2_ref[...]

    hb = BN2 // 2
    _half(pl.ds(0, hb))
    _half(pl.ds(hb, hb))


def kernel(x, Wq, Wk, Wv, Wo, ln1_g, ln1_b, W1, b1, W2, b2, ln2_g, ln2_b):
    bf = jnp.bfloat16
    g1 = ln1_g.reshape(1, D)
    bt1 = ln1_b.reshape(1, D)
    g2 = ln2_g.reshape(1, D)
    bt2 = ln2_b.reshape(1, D)
    bb1 = b1.reshape(1, D_INNER).astype(bf)
    bb2 = b2.reshape(1, D)

    full = lambda shape: pl.BlockSpec(shape, lambda i: (0,) * len(shape))

    kv, ksum, phiq = pl.pallas_call(
        _pass1,
        grid=(N // BN1,),
        in_specs=[
            pl.BlockSpec((BN1, D), lambda i: (i, 0)),
            full((D, D)), full((D, D)), full((D, D)),
            full((1, D)), full((1, D)),
        ],
        out_specs=[full((D, D)), full((1, D)),
                   pl.BlockSpec((BN1, D), lambda i: (i, 0))],
        out_shape=[
            jax.ShapeDtypeStruct((D, D), jnp.float32),
            jax.ShapeDtypeStruct((1, D), jnp.float32),
            jax.ShapeDtypeStruct((N, D), bf),
        ],
    )(x, Wq.astype(bf), Wk.astype(bf), Wv.astype(bf), g1, bt1)

    ksumt = ksum.reshape(D, 1).astype(bf)

    out = pl.pallas_call(
        _pass2,
        grid=(N // BN2,),
        in_specs=[
            pl.BlockSpec((BN2, D), lambda i: (i, 0)),
            pl.BlockSpec((BN2, D), lambda i: (i, 0)),
            full((D, D)), full((D, D)), full((D, 1)),
            full((1, D)), full((1, D)),
            full((D, D_INNER)), full((1, D_INNER)),
            full((D_INNER, D)), full((1, D)),
        ],
        out_specs=pl.BlockSpec((BN2, D), lambda i: (i, 0)),
        out_shape=jax.ShapeDtypeStruct((N, D), jnp.float32),
    )(x, phiq, Wo.astype(bf), kv.astype(bf), ksumt, g2, bt2,
      W1.astype(bf), bb1, W2.astype(bf), bb2)
    return out


# speedup vs baseline: 1.3845x; 1.0534x over previous
import jax
import jax.numpy as jnp
from jax.experimental import pallas as pl
from jax.experimental.pallas import tpu as pltpu

N = 50000
D = 256
D_INNER = 1024
BN = 2000
NB = N // BN
HB = BN // 2


def _phi(z):
    one = jnp.asarray(1.0, z.dtype)
    return jnp.where(z > 0, z + one, jnp.exp(z))


def _ln(xb, g, b, eps=1e-5):
    mu = jnp.mean(xb, axis=-1, keepdims=True)
    ex2 = jnp.mean(xb * xb, axis=-1, keepdims=True)
    var = ex2 - mu * mu
    r = jax.lax.rsqrt(var + eps)
    return (xb - mu) * (r * g) + b


_GC1 = 0.7978845608028654
_GC2 = 0.7978845608028654 * 0.044715


def _gelu(t):
    tt = t * t
    u = t * (jnp.asarray(_GC1, t.dtype) + jnp.asarray(_GC2, t.dtype) * tt)
    th = jnp.tanh(u)
    r = jnp.asarray(0.5, t.dtype) * t
    return r + r * th


def _fused(x_ref, wq_ref, wk_ref, wv_ref, wo_ref, w1_ref, w2_ref,
           g1_ref, b1_ref, g2_ref, b2_ref, bb1_ref, bb2_ref,
           out_ref,
           kv_s, ksumt_s, phiq_s,
           wqb_s, wkb_s, wvb_s, wob_s, w1b_s, w2b_s, bb1b_s):
    i = pl.program_id(0)
    bf = jnp.bfloat16

    @pl.when(i == 0)
    def _init():
        wqb_s[...] = wq_ref[...].astype(bf)
        wkb_s[...] = wk_ref[...].astype(bf)
        wvb_s[...] = wv_ref[...].astype(bf)
        wob_s[...] = wo_ref[...].astype(bf)
        w1b_s[...] = w1_ref[...].astype(bf)
        w2b_s[...] = w2_ref[...].astype(bf)
        bb1b_s[...] = bb1_ref[...].astype(bf)
        kv_s[...] = jnp.zeros_like(kv_s)
        ksumt_s[...] = jnp.zeros_like(ksumt_s)

    @pl.when(i < NB)
    def _phase_a():
        h = _ln(x_ref[...], g1_ref[...], b1_ref[...])
        hb = h.astype(bf)
        q = jnp.dot(hb, wqb_s[...], preferred_element_type=jnp.float32)
        phiq_s[pl.ds(i * BN, BN), :] = _phi(q).astype(bf)
        k = jnp.dot(hb, wkb_s[...], preferred_element_type=jnp.float32)
        v = jnp.dot(hb, wvb_s[...],
                    preferred_element_type=jnp.float32).astype(bf)
        phikb = _phi(k).astype(bf)
        kv_s[...] += jax.lax.dot_general(
            phikb, v, (((0,), (0,)), ((), ())),
            preferred_element_type=jnp.float32)
        ones = jnp.ones((BN, 1), dtype=bf)
        ksumt_s[...] += jax.lax.dot_general(
            phikb, ones, (((0,), (0,)), ((), ())),
            preferred_element_type=jnp.float32)

    @pl.when(i >= NB)
    def _phase_b():
        kvb = kv_s[...].astype(bf)
        ktb = ksumt_s[...].astype(bf)
        j = i - NB

        phiq = phiq_s[pl.ds(j * BN, BN), :]
        num = jnp.dot(phiq, kvb, preferred_element_type=jnp.float32)
        den = jnp.dot(phiq, ktb, preferred_element_type=jnp.float32) + 1e-6
        attn = jnp.dot((num / den).astype(bf), wob_s[...],
                       preferred_element_type=jnp.float32)
        x2 = x_ref[...] + attn

        def _mlp_half(lo, hi):
            x2h = x2[lo:hi]
            h2 = _ln(x2h, g2_ref[...], b2_ref[...])
            t = jnp.dot(h2.astype(bf), w1b_s[...],
                        preferred_element_type=jnp.float32)
            inner = _gelu(t.astype(bf) + bb1b_s[...])
            mlp = jnp.dot(inner, w2b_s[...],
                          preferred_element_type=jnp.float32)
            out_ref[pl.ds(lo, hi - lo), :] = x2h + mlp + bb2_ref[...]

        _mlp_half(0, HB)
        _mlp_half(HB, BN)


def kernel(x, Wq, Wk, Wv, Wo, ln1_g, ln1_b, W1, b1, W2, b2, ln2_g, ln2_b):
    g1 = ln1_g.reshape(1, D)
    bt1 = ln1_b.reshape(1, D)
    g2 = ln2_g.reshape(1, D)
    bt2 = ln2_b.reshape(1, D)
    bb1 = b1.reshape(1, D_INNER)
    bb2 = b2.reshape(1, D)

    full = lambda shape: pl.BlockSpec(shape, lambda i: (0,) * len(shape))
    bf = jnp.bfloat16

    out = pl.pallas_call(
        _fused,
        grid=(2 * NB,),
        in_specs=[
            pl.BlockSpec((BN, D), lambda i: (i % NB, 0)),
            full((D, D)), full((D, D)), full((D, D)), full((D, D)),
            full((D, D_INNER)), full((D_INNER, D)),
            full((1, D)), full((1, D)), full((1, D)), full((1, D)),
            full((1, D_INNER)), full((1, D)),
        ],
        out_specs=pl.BlockSpec(
            (BN, D), lambda i: (jnp.where(i < NB, 0, i - NB), 0)),
        out_shape=jax.ShapeDtypeStruct((N, D), jnp.float32),
        scratch_shapes=[
            pltpu.VMEM((D, D), jnp.float32),
            pltpu.VMEM((D, 1), jnp.float32),
            pltpu.VMEM((N, D), bf),
            pltpu.VMEM((D, D), bf),
            pltpu.VMEM((D, D), bf),
            pltpu.VMEM((D, D), bf),
            pltpu.VMEM((D, D), bf),
            pltpu.VMEM((D, D_INNER), bf),
            pltpu.VMEM((D_INNER, D), bf),
            pltpu.VMEM((1, D_INNER), bf),
        ],
    )(x, Wq, Wk, Wv, Wo, W1, W2, g1, bt1, g2, bt2, bb1, bb2)
    return out


# speedup vs baseline: 1.4782x; 1.0677x over previous
import jax
import jax.numpy as jnp
from jax.experimental import pallas as pl
from jax.experimental.pallas import tpu as pltpu

N = 50000
D = 256
D_INNER = 1024
BN = 2000
NB = N // BN
HB = BN // 2


def _phi(z):
    one = jnp.asarray(1.0, z.dtype)
    return jnp.where(z > 0, z + one, jnp.exp(z))


def _ln(xb, g, b, eps=1e-5):
    mu = jnp.mean(xb, axis=-1, keepdims=True)
    ex2 = jnp.mean(xb * xb, axis=-1, keepdims=True)
    var = ex2 - mu * mu
    r = jax.lax.rsqrt(var + eps)
    return (xb - mu) * (r * g) + b


def _ln_bf16(xb, g, b, eps=1e-5):
    bf = jnp.bfloat16
    mu = jnp.mean(xb, axis=-1, keepdims=True)
    ex2 = jnp.mean(xb * xb, axis=-1, keepdims=True)
    var = ex2 - mu * mu
    r = jax.lax.rsqrt(var + eps)
    return ((xb.astype(bf) - mu.astype(bf)) * (r.astype(bf) * g)
            + b)


_GC1 = 0.7978845608028654
_GC2 = 0.7978845608028654 * 0.044715


def _gelu(t):
    tt = t * t
    u = t * (jnp.asarray(_GC1, t.dtype) + jnp.asarray(_GC2, t.dtype) * tt)
    th = jnp.tanh(u)
    r = jnp.asarray(0.5, t.dtype) * t
    return r + r * th


def _fused(x_ref, wq_ref, wk_ref, wv_ref, wo_ref, w1_ref, w2_ref,
           g1_ref, b1_ref, g2_ref, b2_ref, bb1_ref, bb2_ref,
           out_ref,
           kv_s, ksumt_s, phiq_s,
           wqb_s, wkb_s, wvb_s, wob_s, w1b_s, w2b_s, bb1b_s,
           g1b_s, b1b_s, g2b_s, b2b_s):
    i = pl.program_id(0)
    bf = jnp.bfloat16

    @pl.when(i == 0)
    def _init():
        wqb_s[...] = wq_ref[...].astype(bf)
        wkb_s[...] = wk_ref[...].astype(bf)
        wvb_s[...] = wv_ref[...].astype(bf)
        wob_s[...] = wo_ref[...].astype(bf)
        w1b_s[...] = w1_ref[...].astype(bf)
        w2b_s[...] = w2_ref[...].astype(bf)
        bb1b_s[...] = bb1_ref[...].astype(bf)
        g1b_s[...] = g1_ref[...].astype(bf)
        b1b_s[...] = b1_ref[...].astype(bf)
        g2b_s[...] = g2_ref[...].astype(bf)
        b2b_s[...] = b2_ref[...].astype(bf)
        kv_s[...] = jnp.zeros_like(kv_s)
        ksumt_s[...] = jnp.zeros_like(ksumt_s)

    @pl.when(i < NB)
    def _phase_a():
        hb = _ln_bf16(x_ref[...], g1b_s[...], b1b_s[...])
        q = jnp.dot(hb, wqb_s[...], preferred_element_type=jnp.float32)
        phiq_s[pl.ds(i * BN, BN), :] = _phi(q.astype(bf))
        k = jnp.dot(hb, wkb_s[...], preferred_element_type=jnp.float32)
        v = jnp.dot(hb, wvb_s[...],
                    preferred_element_type=jnp.float32).astype(bf)
        phikb = _phi(k.astype(bf))
        kv_s[...] += jax.lax.dot_general(
            phikb, v, (((0,), (0,)), ((), ())),
            preferred_element_type=jnp.float32)
        ones = jnp.ones((BN, 1), dtype=bf)
        ksumt_s[...] += jax.lax.dot_general(
            phikb, ones, (((0,), (0,)), ((), ())),
            preferred_element_type=jnp.float32)

    @pl.when(i >= NB)
    def _phase_b():
        kvb = kv_s[...].astype(bf)
        ktb = ksumt_s[...].astype(bf)
        j = i - NB

        phiq = phiq_s[pl.ds(j * BN, BN), :]
        num = jnp.dot(phiq, kvb, preferred_element_type=jnp.float32)
        den = jnp.dot(phiq, ktb, preferred_element_type=jnp.float32) + 1e-6
        rden = (1.0 / den).astype(bf)
        attn = jnp.dot(num.astype(bf) * rden, wob_s[...],
                       preferred_element_type=jnp.float32)
        x2 = x_ref[...] + attn

        def _mlp_half(lo, hi):
            x2h = x2[lo:hi]
            h2 = _ln_bf16(x2h, g2b_s[...], b2b_s[...])
            t = jnp.dot(h2, w1b_s[...],
                        preferred_element_type=jnp.float32)
            inner = _gelu(t.astype(bf) + bb1b_s[...])
            mlp = jnp.dot(inner, w2b_s[...],
                          preferred_element_type=jnp.float32)
            out_ref[pl.ds(lo, hi - lo), :] = x2h + mlp + bb2_ref[...]

        _mlp_half(0, HB)
        _mlp_half(HB, BN)


def kernel(x, Wq, Wk, Wv, Wo, ln1_g, ln1_b, W1, b1, W2, b2, ln2_g, ln2_b):
    g1 = ln1_g.reshape(1, D)
    bt1 = ln1_b.reshape(1, D)
    g2 = ln2_g.reshape(1, D)
    bt2 = ln2_b.reshape(1, D)
    bb1 = b1.reshape(1, D_INNER)
    bb2 = b2.reshape(1, D)

    full = lambda shape: pl.BlockSpec(shape, lambda i: (0,) * len(shape))
    bf = jnp.bfloat16

    out = pl.pallas_call(
        _fused,
        grid=(2 * NB,),
        in_specs=[
            pl.BlockSpec((BN, D), lambda i: (i % NB, 0)),
            full((D, D)), full((D, D)), full((D, D)), full((D, D)),
            full((D, D_INNER)), full((D_INNER, D)),
            full((1, D)), full((1, D)), full((1, D)), full((1, D)),
            full((1, D_INNER)), full((1, D)),
        ],
        out_specs=pl.BlockSpec(
            (BN, D), lambda i: (jnp.where(i < NB, 0, i - NB), 0)),
        out_shape=jax.ShapeDtypeStruct((N, D), jnp.float32),
        scratch_shapes=[
            pltpu.VMEM((D, D), jnp.float32),
            pltpu.VMEM((D, 1), jnp.float32),
            pltpu.VMEM((N, D), bf),
            pltpu.VMEM((D, D), bf),
            pltpu.VMEM((D, D), bf),
            pltpu.VMEM((D, D), bf),
            pltpu.VMEM((D, D), bf),
            pltpu.VMEM((D, D_INNER), bf),
            pltpu.VMEM((D_INNER, D), bf),
            pltpu.VMEM((1, D_INNER), bf),
            pltpu.VMEM((1, D), bf),
            pltpu.VMEM((1, D), bf),
            pltpu.VMEM((1, D), bf),
            pltpu.VMEM((1, D), bf),
        ],
    )(x, Wq, Wk, Wv, Wo, W1, W2, g1, bt1, g2, bt2, bb1, bb2)
    return out
